# baseline (speedup 1.0000x reference)
import jax
import jax.numpy as jnp
from jax import lax
from jax.experimental import pallas as pl
from jax.experimental.pallas import tpu as pltpu
from jax.experimental.pallas import tpu_sc as plsc

N = 100000
H = 128
G = 32
NB = G * G
NB1 = NB + 1
R = 2000
NBLK = N // R

CHUNK = 128
NW = 32

SLAB_SIZES = (44000, 36000, 20000)
SLAB_BASES = (0, 44000, 80000)

_mesh = plsc.VectorSubcoreMesh(core_axis_name="c", subcore_axis_name="s")


def _tsig(x):
    return 0.5 * jnp.tanh(0.5 * x) + 0.5


def _lstm_kernel(hid_ref, cell_ref, feat_ref, whh_ref, wf_ref,
                 cnew_ref, hnew_ref):
    hid16 = hid_ref[...].astype(jnp.bfloat16)
    feat_blk = feat_ref[0]
    gates = (jnp.dot(hid16, whh_ref[...], preferred_element_type=jnp.float32)
             + lax.dot_general(feat_blk, wf_ref[...],
                               (((0,), (0,)), ((), ())),
                               preferred_element_type=jnp.float32))
    ii = _tsig(gates[:, :H])
    ff = _tsig(gates[:, H:2 * H])
    gg = jnp.tanh(gates[:, 2 * H:3 * H])
    oo = _tsig(gates[:, 3 * H:])
    c_new = ff * cell_ref[...] + ii * gg
    cnew_ref[...] = c_new
    hnew_ref[...] = oo * jnp.tanh(c_new)


def _make_stage_ids(gbase, m, cpw, span):
    nwf = m // span
    last = m - nwf * span

    def stage(xs_hbm, ys_hbm, xa_v, ya_v, idx_v, wid):
        base = gbase + wid * span

        @pl.when(wid < nwf)
        def _():
            pltpu.sync_copy(xs_hbm.at[pl.ds(base, span)], xa_v)
            pltpu.sync_copy(ys_hbm.at[pl.ds(base, span)], ya_v)

        if last > 0:
            @pl.when(wid == nwf)
            def _():
                pltpu.sync_copy(xs_hbm.at[pl.ds(base, last)],
                                xa_v.at[pl.ds(0, last)])
                pltpu.sync_copy(ys_hbm.at[pl.ds(base, last)],
                                ya_v.at[pl.ds(0, last)])

        @pl.loop(0, cpw)
        def _(j):
            for k in range(CHUNK // 16):
                xs = xa_v[pl.ds(j * CHUNK + 16 * k, 16)]
                ys = ya_v[pl.ds(j * CHUNK + 16 * k, 16)]
                ix = jnp.clip((jnp.clip(xs, 0.0, 1.0) * G).astype(jnp.int32),
                              0, G - 1)
                iy = jnp.clip((jnp.clip(ys, 0.0, 1.0) * G).astype(jnp.int32),
                              0, G - 1)
                ids = ix * G + iy
                row = base + j * CHUNK + 16 * k + lax.iota(jnp.int32, 16)
                idx_v[j, pl.ds(16 * k, 16)] = jnp.where(row < gbase + m,
                                                        ids, NB)

    return stage


def _make_scatter(gbase, m):
    nf = m // CHUNK
    tl = m - nf * CHUNK
    nc = nf + (1 if tl else 0)
    cpw = (nc + NW - 1) // NW
    span = cpw * CHUNK
    stage = _make_stage_ids(gbase, m, cpw, span)

    def _rows_copy(h_hbm, buf_v, wid, j, sem):
        cid = wid * cpw + j
        full = jnp.logical_and(j < cpw, cid < nf)
        tail = jnp.logical_and(j < cpw, cid == nf) if tl else None
        cp_full = pltpu.make_async_copy(
            h_hbm.at[pl.ds(cid * CHUNK, CHUNK), :], buf_v, sem)
        cp_tail = pltpu.make_async_copy(
            h_hbm.at[pl.ds(nf * CHUNK, tl), :],
            buf_v.at[pl.ds(0, tl), :], sem) if tl else None
        return full, tail, cp_full, cp_tail

    def kern(h_hbm, xs_hbm, ys_hbm, zeros_hbm, pbins_hbm,
             rows0_v, rows1_v, idx_v, xa_v, ya_v, bins_sh, sem0, sem1):
        c = lax.axis_index("c")
        s = lax.axis_index("s")
        wid = c * 16 + s
        pltpu.sync_copy(zeros_hbm, bins_sh.at[pl.ds(s * 64, 64), :])
        stage(xs_hbm, ys_hbm, xa_v, ya_v, idx_v, wid)
        plsc.subcore_barrier()

        def start(j, buf, sem):
            full, tail, cp_full, cp_tail = _rows_copy(h_hbm, buf, wid, j, sem)
            pl.when(full)(cp_full.start)
            if tl:
                pl.when(tail)(cp_tail.start)

        def wait(j, buf, sem):
            full, tail, cp_full, cp_tail = _rows_copy(h_hbm, buf, wid, j, sem)
            pl.when(full)(cp_full.wait)
            if tl:
                pl.when(tail)(cp_tail.wait)

        def scatter(j, buf):
            cid = wid * cpw + j

            @pl.when(jnp.logical_and(j < cpw, cid < nc))
            def _():
                pltpu.sync_copy(buf, bins_sh.at[idx_v.at[j]], add=True)

        start(0, rows0_v, sem0)

        @pl.loop(0, (cpw + 1) // 2)
        def _(k):
            j0 = 2 * k
            j1 = 2 * k + 1
            wait(j0, rows0_v, sem0)
            start(j1, rows1_v, sem1)
            scatter(j0, rows0_v)
            wait(j1, rows1_v, sem1)
            start(j0 + 2, rows0_v, sem0)
            scatter(j1, rows1_v)

        plsc.subcore_barrier()
        pltpu.sync_copy(bins_sh.at[pl.ds(s * 64, 64), :],
                        pbins_hbm.at[c, pl.ds(s * 64, 64), :])

    scratch = [
        pltpu.VMEM((CHUNK, H), jnp.float32),
        pltpu.VMEM((CHUNK, H), jnp.float32),
        pltpu.VMEM((cpw, CHUNK), jnp.int32),
        pltpu.VMEM((span,), jnp.float32),
        pltpu.VMEM((span,), jnp.float32),
        pltpu.VMEM_SHARED((NB1, H), jnp.float32),
        pltpu.SemaphoreType.DMA,
        pltpu.SemaphoreType.DMA,
    ]
    return kern, scratch


GNF = N // CHUNK
GTL = N - GNF * CHUNK
GNC = GNF + 1
GCPW = (GNC + NW - 1) // NW
GSPAN = GCPW * CHUNK
_g_stage = _make_stage_ids(0, N, GCPW, GSPAN)


def _sc_gather_kernel(pba_hbm, pbb_hbm, pbc_hbm, xs_hbm, ys_hbm, out_hbm,
                      rows0_v, rows1_v, idx_v, xa_v, ya_v, a_v, b_v, bins_sh,
                      semg0, semg1, semw0, semw1):
    c = lax.axis_index("c")
    s = lax.axis_index("s")
    wid = c * 16 + s
    mb = s * (NB // 16)
    pltpu.sync_copy(pba_hbm.at[0, pl.ds(mb, NB // 16), :], a_v)

    def _accum(src):
        pltpu.sync_copy(src, b_v)

        @pl.loop(0, NB // 16)
        def _(i):
            for j in range(H // 16):
                sl = (i, pl.ds(j * 16, 16))
                a_v[sl] = a_v[sl] + b_v[sl]

    _accum(pba_hbm.at[1, pl.ds(mb, NB // 16), :])
    _accum(pbb_hbm.at[0, pl.ds(mb, NB // 16), :])
    _accum(pbb_hbm.at[1, pl.ds(mb, NB // 16), :])
    _accum(pbc_hbm.at[0, pl.ds(mb, NB // 16), :])
    _accum(pbc_hbm.at[1, pl.ds(mb, NB // 16), :])

    pltpu.sync_copy(a_v, bins_sh.at[pl.ds(mb, NB // 16), :])
    _g_stage(xs_hbm, ys_hbm, xa_v, ya_v, idx_v, wid)
    plsc.subcore_barrier()

    def valid(j):
        return jnp.logical_and(j < GCPW, wid * GCPW + j < GNC)

    def start_g(j, buf, sem):
        cp = pltpu.make_async_copy(bins_sh.at[idx_v.at[j]], buf, sem)
        pl.when(valid(j))(cp.start)

    def wait_g(j, buf, sem):
        cp = pltpu.make_async_copy(bins_sh.at[idx_v.at[j]], buf, sem)
        pl.when(valid(j))(cp.wait)

    def _write_copies(j, buf, sem):
        cid = wid * GCPW + j
        full = jnp.logical_and(j < GCPW, cid < GNF)
        tail = jnp.logical_and(j < GCPW, cid == GNF)
        cp_full = pltpu.make_async_copy(
            buf, out_hbm.at[pl.ds(cid * CHUNK, CHUNK), :], sem)
        cp_tail = pltpu.make_async_copy(
            buf.at[pl.ds(0, GTL), :],
            out_hbm.at[pl.ds(GNF * CHUNK, GTL), :], sem)
        return full, tail, cp_full, cp_tail

    def start_w(j, buf, sem):
        full, tail, cp_full, cp_tail = _write_copies(j, buf, sem)
        pl.when(full)(cp_full.start)
        pl.when(tail)(cp_tail.start)

    def wait_w(j, buf, sem):
        full, tail, cp_full, cp_tail = _write_copies(j, buf, sem)
        pl.when(full)(cp_full.wait)
        pl.when(tail)(cp_tail.wait)

    start_g(0, rows0_v, semg0)
    start_g(1, rows1_v, semg1)

    @pl.loop(0, (GCPW + 1) // 2)
    def _(k):
        j0 = 2 * k
        j1 = 2 * k + 1
        wait_g(j0, rows0_v, semg0)
        start_w(j0, rows0_v, semw0)
        wait_g(j1, rows1_v, semg1)
        start_w(j1, rows1_v, semw1)
        wait_w(j0, rows0_v, semw0)
        start_g(j0 + 2, rows0_v, semg0)
        wait_w(j1, rows1_v, semw1)
        start_g(j1 + 2, rows1_v, semg1)


def kernel(coords, hidden_state, cell_state, W_ih, W_hh, b_ih, b_hh):
    xs = coords[:, 0]
    ys = coords[:, 1]
    whh = W_hh.T.astype(jnp.bfloat16)
    wf = jnp.concatenate(
        [W_ih.T, (b_ih + b_hh)[None, :]], axis=0).astype(jnp.bfloat16)
    feat = jnp.concatenate(
        [xs.reshape(NBLK, 1, R), ys.reshape(NBLK, 1, R),
         jnp.ones((NBLK, 1, R), jnp.float32)],
        axis=1).astype(jnp.bfloat16)
    zeros = jnp.zeros((64, H), jnp.float32)

    def lstm_slab(slab, c_prev=None):
        off = SLAB_BASES[slab] // R
        m = SLAB_SIZES[slab]
        mblk = m // R
        in_specs = [
            pl.BlockSpec((R, H), lambda i: (i + off, 0)),
            pl.BlockSpec((R, H), lambda i: (i + off, 0)),
            pl.BlockSpec((1, 3, R), lambda i: (i + off, 0, 0)),
            pl.BlockSpec((H, 4 * H), lambda i: (0, 0)),
            pl.BlockSpec((3, 4 * H), lambda i: (0, 0)),
        ]
        args = [hidden_state, cell_state, feat, whh, wf]
        aliases = {}
        if c_prev is not None:
            in_specs.append(pl.BlockSpec(memory_space=pl.ANY))
            args.append(c_prev)
            aliases = {5: 0}

        def body(*refs):
            if c_prev is None:
                _lstm_kernel(*refs)
            else:
                hid, cell, ft, w1, w2, _, cn, hn = refs
                _lstm_kernel(hid, cell, ft, w1, w2, cn, hn)

        return pl.pallas_call(
            body,
            grid=(mblk,),
            in_specs=in_specs,
            out_specs=[
                pl.BlockSpec((R, H), lambda i: (i + off, 0)),
                pl.BlockSpec((R, H), lambda i: (i, 0)),
            ],
            out_shape=[
                jax.ShapeDtypeStruct((N, H), jnp.float32),
                jax.ShapeDtypeStruct((m, H), jnp.float32),
            ],
            input_output_aliases=aliases,
        )(*args)

    c0, h0 = lstm_slab(0)
    c1, h1 = lstm_slab(1, c_prev=c0)
    c_new, h2 = lstm_slab(2, c_prev=c1)

    def scatter_slab(slab, h_half):
        kern, scratch = _make_scatter(SLAB_BASES[slab], SLAB_SIZES[slab])
        call = pl.kernel(
            kern,
            out_type=jax.ShapeDtypeStruct((2, NB, H), jnp.float32),
            mesh=_mesh,
            scratch_types=scratch,
        )
        return call(h_half, xs, ys, zeros)

    pb0 = scatter_slab(0, h0)
    pb1 = scatter_slab(1, h1)
    pb2 = scatter_slab(2, h2)

    gather = pl.kernel(
        _sc_gather_kernel,
        out_type=jax.ShapeDtypeStruct((N, H), jnp.float32),
        mesh=_mesh,
        scratch_types=[
            pltpu.VMEM((CHUNK, H), jnp.float32),
            pltpu.VMEM((CHUNK, H), jnp.float32),
            pltpu.VMEM((GCPW, CHUNK), jnp.int32),
            pltpu.VMEM((GSPAN,), jnp.float32),
            pltpu.VMEM((GSPAN,), jnp.float32),
            pltpu.VMEM((NB // 16, H), jnp.float32),
            pltpu.VMEM((NB // 16, H), jnp.float32),
            pltpu.VMEM_SHARED((NB1, H), jnp.float32),
            pltpu.SemaphoreType.DMA,
            pltpu.SemaphoreType.DMA,
            pltpu.SemaphoreType.DMA,
            pltpu.SemaphoreType.DMA,
        ],
    )
    h_social = gather(pb0, pb1, pb2, xs, ys)

    return (h_social, c_new)


# speedup vs baseline: 1.0590x; 1.0590x over previous
import jax
import jax.numpy as jnp
from jax import lax
from jax.experimental import pallas as pl
from jax.experimental.pallas import tpu as pltpu
from jax.experimental.pallas import tpu_sc as plsc

N = 100000
H = 128
G = 32
NB = G * G
NB1 = NB + 1
R = 2000
NBLK = N // R

CHUNK = 128
NW = 32

SLABS = 2
M = N // SLABS
MBLK = M // R

_mesh = plsc.VectorSubcoreMesh(core_axis_name="c", subcore_axis_name="s")


def _tsig(x):
    return 0.5 * jnp.tanh(0.5 * x) + 0.5


def _lstm_kernel(hid_ref, cell_ref, feat_ref, whh_ref, wf_ref,
                 cnew_ref, hnew_ref):
    hid16 = hid_ref[...].astype(jnp.bfloat16)
    feat_blk = feat_ref[0]
    gates = (jnp.dot(hid16, whh_ref[...], preferred_element_type=jnp.float32)
             + lax.dot_general(feat_blk, wf_ref[...],
                               (((0,), (0,)), ((), ())),
                               preferred_element_type=jnp.float32))
    ii = _tsig(gates[:, :H])
    ff = _tsig(gates[:, H:2 * H])
    gg = jnp.tanh(gates[:, 2 * H:3 * H])
    oo = _tsig(gates[:, 3 * H:])
    c_new = ff * cell_ref[...] + ii * gg
    cnew_ref[...] = c_new
    hnew_ref[...] = oo * jnp.tanh(c_new)


def _make_stage_ids(gbase, m, cpw, span):
    nwf = m // span
    last = m - nwf * span

    def stage(xs_hbm, ys_hbm, xa_v, ya_v, idx_v, wid):
        base = gbase + wid * span

        @pl.when(wid < nwf)
        def _():
            pltpu.sync_copy(xs_hbm.at[pl.ds(base, span)], xa_v)
            pltpu.sync_copy(ys_hbm.at[pl.ds(base, span)], ya_v)

        if last > 0:
            @pl.when(wid == nwf)
            def _():
                pltpu.sync_copy(xs_hbm.at[pl.ds(base, last)],
                                xa_v.at[pl.ds(0, last)])
                pltpu.sync_copy(ys_hbm.at[pl.ds(base, last)],
                                ya_v.at[pl.ds(0, last)])

        @pl.loop(0, cpw)
        def _(j):
            for k in range(CHUNK // 16):
                xs = xa_v[pl.ds(j * CHUNK + 16 * k, 16)]
                ys = ya_v[pl.ds(j * CHUNK + 16 * k, 16)]
                ix = jnp.clip((jnp.clip(xs, 0.0, 1.0) * G).astype(jnp.int32),
                              0, G - 1)
                iy = jnp.clip((jnp.clip(ys, 0.0, 1.0) * G).astype(jnp.int32),
                              0, G - 1)
                ids = ix * G + iy
                row = base + j * CHUNK + 16 * k + lax.iota(jnp.int32, 16)
                idx_v[j, pl.ds(16 * k, 16)] = jnp.where(row < gbase + m,
                                                        ids, NB)

    return stage


def _make_scatter(gbase, m):
    nf = m // CHUNK
    tl = m - nf * CHUNK
    nc = nf + (1 if tl else 0)
    cpw = (nc + NW - 1) // NW
    span = cpw * CHUNK
    stage = _make_stage_ids(gbase, m, cpw, span)

    def _rows_copy(h_hbm, buf_v, wid, j, sem):
        cid = wid * cpw + j
        full = jnp.logical_and(j < cpw, cid < nf)
        tail = jnp.logical_and(j < cpw, cid == nf) if tl else None
        cp_full = pltpu.make_async_copy(
            h_hbm.at[pl.ds(cid * CHUNK, CHUNK), :], buf_v, sem)
        cp_tail = pltpu.make_async_copy(
            h_hbm.at[pl.ds(nf * CHUNK, tl), :],
            buf_v.at[pl.ds(0, tl), :], sem) if tl else None
        return full, tail, cp_full, cp_tail

    def kern(h_hbm, xs_hbm, ys_hbm, zeros_hbm, pbins_hbm,
             rows0_v, rows1_v, idx_v, xa_v, ya_v, bins_sh, sem0, sem1):
        c = lax.axis_index("c")
        s = lax.axis_index("s")
        wid = c * 16 + s
        pltpu.sync_copy(zeros_hbm, bins_sh.at[pl.ds(s * 64, 64), :])
        stage(xs_hbm, ys_hbm, xa_v, ya_v, idx_v, wid)
        plsc.subcore_barrier()

        def start(j, buf, sem):
            full, tail, cp_full, cp_tail = _rows_copy(h_hbm, buf, wid, j, sem)
            pl.when(full)(cp_full.start)
            if tl:
                pl.when(tail)(cp_tail.start)

        def wait(j, buf, sem):
            full, tail, cp_full, cp_tail = _rows_copy(h_hbm, buf, wid, j, sem)
            pl.when(full)(cp_full.wait)
            if tl:
                pl.when(tail)(cp_tail.wait)

        def scatter(j, buf):
            cid = wid * cpw + j

            @pl.when(jnp.logical_and(j < cpw, cid < nc))
            def _():
                pltpu.sync_copy(buf, bins_sh.at[idx_v.at[j]], add=True)

        start(0, rows0_v, sem0)

        @pl.loop(0, (cpw + 1) // 2)
        def _(k):
            j0 = 2 * k
            j1 = 2 * k + 1
            wait(j0, rows0_v, sem0)
            start(j1, rows1_v, sem1)
            scatter(j0, rows0_v)
            wait(j1, rows1_v, sem1)
            start(j0 + 2, rows0_v, sem0)
            scatter(j1, rows1_v)

        plsc.subcore_barrier()
        pltpu.sync_copy(bins_sh.at[pl.ds(s * 64, 64), :],
                        pbins_hbm.at[c, pl.ds(s * 64, 64), :])

    scratch = [
        pltpu.VMEM((CHUNK, H), jnp.float32),
        pltpu.VMEM((CHUNK, H), jnp.float32),
        pltpu.VMEM((cpw, CHUNK), jnp.int32),
        pltpu.VMEM((span,), jnp.float32),
        pltpu.VMEM((span,), jnp.float32),
        pltpu.VMEM_SHARED((NB1, H), jnp.float32),
        pltpu.SemaphoreType.DMA,
        pltpu.SemaphoreType.DMA,
    ]
    return kern, scratch


GNF = N // CHUNK
GTL = N - GNF * CHUNK
GNC = GNF + 1
GCPW = (GNC + NW - 1) // NW
GSPAN = GCPW * CHUNK
_g_stage = _make_stage_ids(0, N, GCPW, GSPAN)


def _sc_gather_kernel(pba_hbm, pbb_hbm, xs_hbm, ys_hbm, out_hbm,
                      rows0_v, rows1_v, rows2_v, rows3_v,
                      idx_v, xa_v, ya_v, a_v, b_v, bins_sh,
                      semg0, semg1, semg2, semg3, semw0, semw1, semw2, semw3):
    c = lax.axis_index("c")
    s = lax.axis_index("s")
    wid = c * 16 + s
    mb = s * (NB // 16)
    pltpu.sync_copy(pba_hbm.at[0, pl.ds(mb, NB // 16), :], a_v)

    def _accum(src):
        pltpu.sync_copy(src, b_v)

        @pl.loop(0, NB // 16)
        def _(i):
            for j in range(H // 16):
                sl = (i, pl.ds(j * 16, 16))
                a_v[sl] = a_v[sl] + b_v[sl]

    _accum(pba_hbm.at[1, pl.ds(mb, NB // 16), :])
    _accum(pbb_hbm.at[0, pl.ds(mb, NB // 16), :])
    _accum(pbb_hbm.at[1, pl.ds(mb, NB // 16), :])

    pltpu.sync_copy(a_v, bins_sh.at[pl.ds(mb, NB // 16), :])
    _g_stage(xs_hbm, ys_hbm, xa_v, ya_v, idx_v, wid)
    plsc.subcore_barrier()

    def valid(j):
        return jnp.logical_and(j < GCPW, wid * GCPW + j < GNC)

    def start_g(j, buf, sem):
        cp = pltpu.make_async_copy(bins_sh.at[idx_v.at[j]], buf, sem)
        pl.when(valid(j))(cp.start)

    def wait_g(j, buf, sem):
        cp = pltpu.make_async_copy(bins_sh.at[idx_v.at[j]], buf, sem)
        pl.when(valid(j))(cp.wait)

    def _write_copies(j, buf, sem):
        cid = wid * GCPW + j
        full = jnp.logical_and(j < GCPW, cid < GNF)
        tail = jnp.logical_and(j < GCPW, cid == GNF)
        cp_full = pltpu.make_async_copy(
            buf, out_hbm.at[pl.ds(cid * CHUNK, CHUNK), :], sem)
        cp_tail = pltpu.make_async_copy(
            buf.at[pl.ds(0, GTL), :],
            out_hbm.at[pl.ds(GNF * CHUNK, GTL), :], sem)
        return full, tail, cp_full, cp_tail

    def start_w(j, buf, sem):
        full, tail, cp_full, cp_tail = _write_copies(j, buf, sem)
        pl.when(full)(cp_full.start)
        pl.when(tail)(cp_tail.start)

    def wait_w(j, buf, sem):
        full, tail, cp_full, cp_tail = _write_copies(j, buf, sem)
        pl.when(full)(cp_full.wait)
        pl.when(tail)(cp_tail.wait)

    bufs = (rows0_v, rows1_v, rows2_v, rows3_v)
    semsg = (semg0, semg1, semg2, semg3)
    semsw = (semw0, semw1, semw2, semw3)
    for t in range(4):
        start_g(t, bufs[t], semsg[t])

    @pl.loop(0, (GCPW + 3) // 4)
    def _(k):
        for t in range(4):
            j = 4 * k + t
            wait_g(j, bufs[t], semsg[t])
            start_w(j, bufs[t], semsw[t])
        for t in range(4):
            j = 4 * k + t
            wait_w(j, bufs[t], semsw[t])
            start_g(j + 4, bufs[t], semsg[t])


def kernel(coords, hidden_state, cell_state, W_ih, W_hh, b_ih, b_hh):
    xs = coords[:, 0]
    ys = coords[:, 1]
    whh = W_hh.T.astype(jnp.bfloat16)
    wf = jnp.concatenate(
        [W_ih.T, (b_ih + b_hh)[None, :]], axis=0).astype(jnp.bfloat16)
    feat = jnp.concatenate(
        [xs.reshape(NBLK, 1, R), ys.reshape(NBLK, 1, R),
         jnp.ones((NBLK, 1, R), jnp.float32)],
        axis=1).astype(jnp.bfloat16)
    zeros = jnp.zeros((64, H), jnp.float32)

    def lstm_slab(slab, c_prev=None):
        off = slab * MBLK
        in_specs = [
            pl.BlockSpec((R, H), lambda i: (i + off, 0)),
            pl.BlockSpec((R, H), lambda i: (i + off, 0)),
            pl.BlockSpec((1, 3, R), lambda i: (i + off, 0, 0)),
            pl.BlockSpec((H, 4 * H), lambda i: (0, 0)),
            pl.BlockSpec((3, 4 * H), lambda i: (0, 0)),
        ]
        args = [hidden_state, cell_state, feat, whh, wf]
        aliases = {}
        if c_prev is not None:
            in_specs.append(pl.BlockSpec(memory_space=pl.ANY))
            args.append(c_prev)
            aliases = {5: 0}

        def body(*refs):
            if c_prev is None:
                _lstm_kernel(*refs)
            else:
                hid, cell, ft, w1, w2, _, cn, hn = refs
                _lstm_kernel(hid, cell, ft, w1, w2, cn, hn)

        return pl.pallas_call(
            body,
            grid=(MBLK,),
            in_specs=in_specs,
            out_specs=[
                pl.BlockSpec((R, H), lambda i: (i + off, 0)),
                pl.BlockSpec((R, H), lambda i: (i, 0)),
            ],
            out_shape=[
                jax.ShapeDtypeStruct((N, H), jnp.float32),
                jax.ShapeDtypeStruct((M, H), jnp.float32),
            ],
            input_output_aliases=aliases,
        )(*args)

    c0, h0 = lstm_slab(0)
    c_new, h1 = lstm_slab(1, c_prev=c0)

    def scatter_slab(slab, h_half):
        kern, scratch = _make_scatter(slab * M, M)
        call = pl.kernel(
            kern,
            out_type=jax.ShapeDtypeStruct((2, NB, H), jnp.float32),
            mesh=_mesh,
            scratch_types=scratch,
        )
        return call(h_half, xs, ys, zeros)

    pb0 = scatter_slab(0, h0)
    pb1 = scatter_slab(1, h1)

    gather = pl.kernel(
        _sc_gather_kernel,
        out_type=jax.ShapeDtypeStruct((N, H), jnp.float32),
        mesh=_mesh,
        scratch_types=[
            pltpu.VMEM((CHUNK, H), jnp.float32),
            pltpu.VMEM((CHUNK, H), jnp.float32),
            pltpu.VMEM((CHUNK, H), jnp.float32),
            pltpu.VMEM((CHUNK, H), jnp.float32),
            pltpu.VMEM((GCPW, CHUNK), jnp.int32),
            pltpu.VMEM((GSPAN,), jnp.float32),
            pltpu.VMEM((GSPAN,), jnp.float32),
            pltpu.VMEM((NB // 16, H), jnp.float32),
            pltpu.VMEM((NB // 16, H), jnp.float32),
            pltpu.VMEM_SHARED((NB1, H), jnp.float32),
        ] + [pltpu.SemaphoreType.DMA] * 8,
    )
    h_social = gather(pb0, pb1, xs, ys)

    return (h_social, c_new)
